# baseline (speedup 1.0000x reference)
import jax
import jax.numpy as jnp
from jax.experimental import pallas as pl
from jax.experimental.pallas import tpu as pltpu

N, D = 16384, 512
H, C = 128, 32
K = 1024
BN = 4096
NB = N // BN
BN2 = 512
EPS = 1e-5
INV_N = 1.0 / N


def _mm(a, b):
    return jnp.dot(a.astype(jnp.bfloat16), b.astype(jnp.bfloat16),
                   preferred_element_type=jnp.float32)


def _body(X_ref, W1_ref, b1_ref, g1_ref, be1_ref, W2_ref, b2_ref,
          dW1_ref, db1_ref, dg1_ref, dbe1_ref, dW2_ref, dW2T_ref, db2_ref,
          cb_ref, cbT_ref, b2row_ref,
          topics_ref, loss_ref,
          h1_scr, h2_scr, y_scr, mw_scr,
          s1, v1, s2, v2, cnt, sx, sxx, zl, cross):
    p = pl.program_id(0)
    b = pl.program_id(1)
    rows = pl.ds(b * BN, BN)

    @pl.when(p == 0)
    def _p0():
        x = X_ref[...]
        h = _mm(x, W1_ref[...]) + b1_ref[...]
        h1_scr[rows, :] = h
        y_scr[rows, :] = _mm(x, dW2T_ref[...]).astype(jnp.bfloat16)
        blk = jnp.sum(h, axis=0, keepdims=True)
        blk_sx = jnp.sum(x, axis=0, keepdims=True)
        blk_sxx = jnp.sum(jnp.sum(x * x, axis=1, keepdims=True), axis=0,
                          keepdims=True)

        @pl.when(b == 0)
        def _():
            s1[...] = blk
            sx[...] = blk_sx
            sxx[...] = blk_sxx

        @pl.when(b != 0)
        def _():
            s1[...] += blk
            sx[...] += blk_sx
            sxx[...] += blk_sxx

    @pl.when(p == 1)
    def _p1():
        mu = s1[...] * INV_N
        d = h1_scr[rows, :] - mu
        blk = jnp.sum(d * d, axis=0, keepdims=True)

        @pl.when(b == 0)
        def _():
            v1[...] = blk
            mw_scr[...] = _mm(cb_ref[...], dW1_ref[...]) + db1_ref[...]

        @pl.when(b != 0)
        def _():
            v1[...] += blk

    @pl.when(p == 2)
    def _p2():
        mu = s1[...] * INV_N
        sd = jnp.sqrt(v1[...] * INV_N + EPS)
        mwb = mw_scr[...].astype(jnp.bfloat16)

        def _chunk(base):
            crows = pl.ds(base, BN2)
            t = (h1_scr[crows, :] - mu) / sd * g1_ref[...] + be1_ref[...]
            r = jnp.maximum(t, 0.0)
            z = _mm(r, W2_ref[...]) + b2_ref[...]
            a2 = jnp.sum(z * z, axis=1, keepdims=True)
            ab2 = _mm(z + z, cbT_ref[...])
            dist = (a2 - ab2) + b2row_ref[...]
            mn = jnp.min(dist, axis=1, keepdims=True)
            iota = jax.lax.broadcasted_iota(jnp.int32, (BN2, K), 1)
            eq = dist == mn
            am = jnp.min(jnp.where(eq, iota, K), axis=1, keepdims=True)
            topics_ref[crows, :] = am
            ohf = jnp.where(eq, jnp.float32(1), jnp.float32(0))
            c_cnt = jnp.sum(ohf, axis=0, keepdims=True)
            oh = ohf.astype(jnp.bfloat16)
            h2 = jnp.dot(oh, mwb, preferred_element_type=jnp.float32)
            h2_scr[crows, :] = h2.astype(jnp.bfloat16)
            return (jnp.sum(mn, axis=0, keepdims=True), c_cnt,
                    jnp.sum(h2, axis=0, keepdims=True),
                    jnp.sum(h2 * h2, axis=0, keepdims=True))

        parts = [_chunk(b * BN + k * BN2) for k in range(BN // BN2)]
        blk_zl = sum((x[0] for x in parts[1:]), parts[0][0])
        blk_cnt = sum((x[1] for x in parts[1:]), parts[0][1])
        blk_s2 = sum((x[2] for x in parts[1:]), parts[0][2])
        blk_v2 = sum((x[3] for x in parts[1:]), parts[0][3])

        @pl.when(b == 0)
        def _():
            zl[...] = blk_zl
            cnt[...] = blk_cnt
            s2[...] = blk_s2
            v2[...] = blk_v2

        @pl.when(b != 0)
        def _():
            zl[...] += blk_zl
            cnt[...] += blk_cnt
            s2[...] += blk_s2
            v2[...] += blk_v2

    @pl.when(p == 3)
    def _p3():
        mu = s2[...] * INV_N
        sd = jnp.sqrt((v2[...] * INV_N - mu * mu) + EPS)
        r2 = jnp.maximum(
            (h2_scr[rows, :].astype(jnp.float32) - mu) / sd * dg1_ref[...]
            + dbe1_ref[...], 0.0)
        yf = y_scr[rows, :].astype(jnp.float32)
        blk = jnp.sum(jnp.sum(r2 * yf, axis=1, keepdims=True), axis=0,
                      keepdims=True)

        @pl.when(b == 0)
        def _():
            cross[...] = blk

        @pl.when(b != 0)
        def _():
            cross[...] += blk

        @pl.when(b == NB - 1)
        def _():
            pt = jnp.maximum(
                (mw_scr[...] - mu) / sd * dg1_ref[...] + dbe1_ref[...], 0.0)
            xc = _mm(pt, dW2_ref[...]) + db2_ref[...]
            w = xc * xc
            scn_row = jnp.dot(cnt[...], w, preferred_element_type=jnp.float32)
            scn = jnp.sum(scn_row, axis=1, keepdims=True)
            cx = jnp.sum(db2_ref[...] * sx[...], axis=1, keepdims=True)
            se = (scn - 2.0 * (cross[...] + cx)) + sxx[...]
            loss_ref[...] = (zl[...] + zl[...]) + jnp.sqrt(se)


def _x_index(p, b):
    return (jax.lax.select(p == 0, b, 0), 0)


def _const(p, b):
    return (0, 0)


def kernel(X, enc_W1, enc_b1, enc_g1, enc_be1, enc_W2, enc_b2,
           dec_W1, dec_b1, dec_g1, dec_be1, dec_W2, dec_b2, codebook):
    f32 = jnp.float32
    b2row = jnp.sum(codebook * codebook, axis=1).reshape(1, K)
    cbT = codebook.T
    dW2T = dec_W2.T

    row = lambda v: v.reshape(1, -1)
    in_specs = [
        pl.BlockSpec((BN, D), _x_index),
        pl.BlockSpec((D, H), _const),
        pl.BlockSpec((1, H), _const),
        pl.BlockSpec((1, H), _const),
        pl.BlockSpec((1, H), _const),
        pl.BlockSpec((H, C), _const),
        pl.BlockSpec((1, C), _const),
        pl.BlockSpec((C, H), _const),
        pl.BlockSpec((1, H), _const),
        pl.BlockSpec((1, H), _const),
        pl.BlockSpec((1, H), _const),
        pl.BlockSpec((H, D), _const),
        pl.BlockSpec((D, H), _const),
        pl.BlockSpec((1, D), _const),
        pl.BlockSpec((K, C), _const),
        pl.BlockSpec((C, K), _const),
        pl.BlockSpec((1, K), _const),
    ]
    out_specs = [
        pl.BlockSpec((N, 1), _const),
        pl.BlockSpec((1, 1), _const),
    ]
    topics2d, loss2d = pl.pallas_call(
        _body,
        grid=(4, NB),
        in_specs=in_specs,
        out_specs=out_specs,
        out_shape=[
            jax.ShapeDtypeStruct((N, 1), jnp.int32),
            jax.ShapeDtypeStruct((1, 1), f32),
        ],
        scratch_shapes=[
            pltpu.VMEM((N, H), f32),
            pltpu.VMEM((N, H), jnp.bfloat16),
            pltpu.VMEM((N, H), jnp.bfloat16),
            pltpu.VMEM((K, H), f32),
            pltpu.VMEM((1, H), f32),
            pltpu.VMEM((1, H), f32),
            pltpu.VMEM((1, H), f32),
            pltpu.VMEM((1, H), f32),
            pltpu.VMEM((1, K), f32),
            pltpu.VMEM((1, D), f32),
            pltpu.VMEM((1, 1), f32),
            pltpu.VMEM((1, 1), f32),
            pltpu.VMEM((1, 1), f32),
        ],
    )(X, enc_W1, row(enc_b1), row(enc_g1), row(enc_be1), enc_W2,
      row(enc_b2), dec_W1, row(dec_b1), row(dec_g1), row(dec_be1), dec_W2,
      dW2T, row(dec_b2), codebook, cbT, b2row)
    return topics2d.reshape(N), loss2d[0, 0]


# speedup vs baseline: 1.0181x; 1.0181x over previous
import jax
import jax.numpy as jnp
from jax.experimental import pallas as pl
from jax.experimental.pallas import tpu as pltpu

N, D = 16384, 512
H, C = 128, 32
K = 1024
BN = 4096
NB = N // BN
BN2 = 1024
EPS = 1e-5
INV_N = 1.0 / N


def _mm(a, b):
    return jnp.dot(a.astype(jnp.bfloat16), b.astype(jnp.bfloat16),
                   preferred_element_type=jnp.float32)


def _body(X_ref, W1_ref, b1_ref, g1_ref, be1_ref, W2_ref, b2_ref,
          dW1_ref, db1_ref, dg1_ref, dbe1_ref, dW2_ref, dW2T_ref, db2_ref,
          cb_ref, cbT_ref, b2row_ref,
          topics_ref, loss_ref,
          h1_scr, h2_scr, y_scr, mw_scr,
          s1, v1, s2, v2, cnt, sx, sxx, zl, cross):
    p = pl.program_id(0)
    b = pl.program_id(1)
    rows = pl.ds(b * BN, BN)

    @pl.when(p == 0)
    def _p0():
        x = X_ref[...]
        h = _mm(x, W1_ref[...]) + b1_ref[...]
        h1_scr[rows, :] = h
        y_scr[rows, :] = _mm(x, dW2T_ref[...]).astype(jnp.bfloat16)
        blk = jnp.sum(h, axis=0, keepdims=True)
        blk_sx = jnp.sum(x, axis=0, keepdims=True)
        blk_sxx = jnp.sum(jnp.sum(x * x, axis=1, keepdims=True), axis=0,
                          keepdims=True)

        @pl.when(b == 0)
        def _():
            s1[...] = blk
            sx[...] = blk_sx
            sxx[...] = blk_sxx

        @pl.when(b != 0)
        def _():
            s1[...] += blk
            sx[...] += blk_sx
            sxx[...] += blk_sxx

    @pl.when(p == 1)
    def _p1():
        mu = s1[...] * INV_N
        d = h1_scr[rows, :] - mu
        blk = jnp.sum(d * d, axis=0, keepdims=True)

        @pl.when(b == 0)
        def _():
            v1[...] = blk
            mw_scr[...] = _mm(cb_ref[...], dW1_ref[...]) + db1_ref[...]

        @pl.when(b != 0)
        def _():
            v1[...] += blk

    @pl.when(p == 2)
    def _p2():
        mu = s1[...] * INV_N
        sd = jnp.sqrt(v1[...] * INV_N + EPS)
        mwb = mw_scr[...].astype(jnp.bfloat16)

        def _chunk(base):
            crows = pl.ds(base, BN2)
            t = (h1_scr[crows, :] - mu) / sd * g1_ref[...] + be1_ref[...]
            r = jnp.maximum(t, 0.0)
            z = _mm(r, W2_ref[...]) + b2_ref[...]
            a2 = jnp.sum(z * z, axis=1, keepdims=True)
            ab2 = _mm(z + z, cbT_ref[...])
            dist = (a2 - ab2) + b2row_ref[...]
            mn = jnp.min(dist, axis=1, keepdims=True)
            iota = jax.lax.broadcasted_iota(jnp.int32, (BN2, K), 1)
            eq = dist == mn
            am = jnp.min(jnp.where(eq, iota, K), axis=1, keepdims=True)
            topics_ref[crows, :] = am
            ohf = jnp.where(eq, jnp.float32(1), jnp.float32(0))
            c_cnt = jnp.sum(ohf, axis=0, keepdims=True)
            oh = ohf.astype(jnp.bfloat16)
            h2 = jnp.dot(oh, mwb, preferred_element_type=jnp.float32)
            h2_scr[crows, :] = h2.astype(jnp.bfloat16)
            return (jnp.sum(mn, axis=0, keepdims=True), c_cnt,
                    jnp.sum(h2, axis=0, keepdims=True),
                    jnp.sum(h2 * h2, axis=0, keepdims=True))

        parts = [_chunk(b * BN + k * BN2) for k in range(BN // BN2)]
        blk_zl = sum((x[0] for x in parts[1:]), parts[0][0])
        blk_cnt = sum((x[1] for x in parts[1:]), parts[0][1])
        blk_s2 = sum((x[2] for x in parts[1:]), parts[0][2])
        blk_v2 = sum((x[3] for x in parts[1:]), parts[0][3])

        @pl.when(b == 0)
        def _():
            zl[...] = blk_zl
            cnt[...] = blk_cnt
            s2[...] = blk_s2
            v2[...] = blk_v2

        @pl.when(b != 0)
        def _():
            zl[...] += blk_zl
            cnt[...] += blk_cnt
            s2[...] += blk_s2
            v2[...] += blk_v2

    @pl.when(p == 3)
    def _p3():
        mu = s2[...] * INV_N
        sd = jnp.sqrt((v2[...] * INV_N - mu * mu) + EPS)
        r2 = jnp.maximum(
            (h2_scr[rows, :].astype(jnp.float32) - mu) / sd * dg1_ref[...]
            + dbe1_ref[...], 0.0)
        yf = y_scr[rows, :].astype(jnp.float32)
        blk = jnp.sum(jnp.sum(r2 * yf, axis=1, keepdims=True), axis=0,
                      keepdims=True)

        @pl.when(b == 0)
        def _():
            cross[...] = blk

        @pl.when(b != 0)
        def _():
            cross[...] += blk

        @pl.when(b == NB - 1)
        def _():
            pt = jnp.maximum(
                (mw_scr[...] - mu) / sd * dg1_ref[...] + dbe1_ref[...], 0.0)
            xc = _mm(pt, dW2_ref[...]) + db2_ref[...]
            w = xc * xc
            scn_row = jnp.dot(cnt[...], w, preferred_element_type=jnp.float32)
            scn = jnp.sum(scn_row, axis=1, keepdims=True)
            cx = jnp.sum(db2_ref[...] * sx[...], axis=1, keepdims=True)
            se = (scn - 2.0 * (cross[...] + cx)) + sxx[...]
            loss_ref[...] = (zl[...] + zl[...]) + jnp.sqrt(se)


def _x_index(p, b):
    return (jax.lax.select(p == 0, b, 0), 0)


def _const(p, b):
    return (0, 0)


def kernel(X, enc_W1, enc_b1, enc_g1, enc_be1, enc_W2, enc_b2,
           dec_W1, dec_b1, dec_g1, dec_be1, dec_W2, dec_b2, codebook):
    f32 = jnp.float32
    b2row = jnp.sum(codebook * codebook, axis=1).reshape(1, K)
    cbT = codebook.T
    dW2T = dec_W2.T

    row = lambda v: v.reshape(1, -1)
    in_specs = [
        pl.BlockSpec((BN, D), _x_index),
        pl.BlockSpec((D, H), _const),
        pl.BlockSpec((1, H), _const),
        pl.BlockSpec((1, H), _const),
        pl.BlockSpec((1, H), _const),
        pl.BlockSpec((H, C), _const),
        pl.BlockSpec((1, C), _const),
        pl.BlockSpec((C, H), _const),
        pl.BlockSpec((1, H), _const),
        pl.BlockSpec((1, H), _const),
        pl.BlockSpec((1, H), _const),
        pl.BlockSpec((H, D), _const),
        pl.BlockSpec((D, H), _const),
        pl.BlockSpec((1, D), _const),
        pl.BlockSpec((K, C), _const),
        pl.BlockSpec((C, K), _const),
        pl.BlockSpec((1, K), _const),
    ]
    out_specs = [
        pl.BlockSpec((N, 1), _const),
        pl.BlockSpec((1, 1), _const),
    ]
    topics2d, loss2d = pl.pallas_call(
        _body,
        grid=(4, NB),
        in_specs=in_specs,
        out_specs=out_specs,
        out_shape=[
            jax.ShapeDtypeStruct((N, 1), jnp.int32),
            jax.ShapeDtypeStruct((1, 1), f32),
        ],
        scratch_shapes=[
            pltpu.VMEM((N, H), f32),
            pltpu.VMEM((N, H), jnp.bfloat16),
            pltpu.VMEM((N, H), jnp.bfloat16),
            pltpu.VMEM((K, H), f32),
            pltpu.VMEM((1, H), f32),
            pltpu.VMEM((1, H), f32),
            pltpu.VMEM((1, H), f32),
            pltpu.VMEM((1, H), f32),
            pltpu.VMEM((1, K), f32),
            pltpu.VMEM((1, D), f32),
            pltpu.VMEM((1, 1), f32),
            pltpu.VMEM((1, 1), f32),
            pltpu.VMEM((1, 1), f32),
        ],
    )(X, enc_W1, row(enc_b1), row(enc_g1), row(enc_be1), enc_W2,
      row(enc_b2), dec_W1, row(dec_b1), row(dec_g1), row(dec_be1), dec_W2,
      dW2T, row(dec_b2), codebook, cbT, b2row)
    return topics2d.reshape(N), loss2d[0, 0]
